# baseline (speedup 1.0000x reference)
import jax
import jax.numpy as jnp
from jax import lax
from jax.experimental import pallas as pl
from jax.experimental.pallas import tpu as pltpu
from jax.experimental.pallas import tpu_sc as plsc

_INDIM = 2048
_OUTDIM = 2048
_FANIN = 32
_NTOK = 2048

_E = _OUTDIM * _FANIN
_NUM_WORKERS = 32
_NHALF = 2
_HALF_ROWS = _OUTDIM // _NHALF
_ROWS_PER_WORKER = _HALF_ROWS // _NUM_WORKERS
_CHUNK_ROWS = 16
_CHUNK_EDGES = _CHUNK_ROWS * _FANIN
_LANES = 16
_NBUF = 2


def _scatter_body(ei_hbm, w_hbm, wout_hbm, wbufs, srcbuf, wvbuf, wb16, sems,
                  *, row0):
    wid = lax.axis_index("s") * 2 + lax.axis_index("c")
    iota = lax.iota(jnp.int32, _LANES)
    nchunks = _ROWS_PER_WORKER // _CHUNK_ROWS
    pending = [None] * _NBUF

    zeros16 = jnp.zeros((_LANES,), jnp.float32)
    for buf in range(_NBUF):
        for r in range(_CHUNK_ROWS):
            def _zcol(j, carry, buf=buf, r=r):
                base = j * (_LANES * 8)
                for u in range(8):
                    wbufs[buf, r, pl.ds(base + u * _LANES, _LANES)] = zeros16
                return carry
            lax.fori_loop(0, _INDIM // (_LANES * 8), _zcol, 0)

    for chunk in range(nchunks):
        buf = chunk % _NBUF
        row_local = wid * _ROWS_PER_WORKER + chunk * _CHUNK_ROWS
        edge_base = (row0 + row_local) * _FANIN
        wbuf = wbufs.at[buf]

        if pending[buf] is not None:
            pending[buf].wait()
            pending[buf] = None
            for k in range(_FANIN):
                le = iota * _FANIN + (buf * _CHUNK_EDGES + k)
                src_vec = plsc.load_gather(srcbuf, [le])
                w_vec = plsc.load_gather(wvbuf, [le])
                plsc.addupdate_scatter(wbuf, [iota, src_vec], -w_vec)

        pltpu.sync_copy(ei_hbm.at[pl.ds(_E + edge_base, _CHUNK_EDGES)],
                        srcbuf.at[pl.ds(buf * _CHUNK_EDGES, _CHUNK_EDGES)])
        pltpu.sync_copy(w_hbm.at[pl.ds(edge_base, _CHUNK_EDGES)],
                        wvbuf.at[pl.ds(buf * _CHUNK_EDGES, _CHUNK_EDGES)])

        for k in range(_FANIN):
            le = iota * _FANIN + (buf * _CHUNK_EDGES + k)
            src_vec = plsc.load_gather(srcbuf, [le])
            w_vec = plsc.load_gather(wvbuf, [le])
            plsc.addupdate_scatter(wbuf, [iota, src_vec], w_vec)

        for r in range(_CHUNK_ROWS):
            def _pk(j, carry, buf=buf, r=r):
                base = j * (8 * _LANES)
                for u in range(4):
                    off = base + u * 2 * _LANES
                    a = wbufs[buf, r, pl.ds(off, _LANES)]
                    b = wbufs[buf, r, pl.ds(off + _LANES, _LANES)]
                    packed = plsc.pack(
                        a, b, format=plsc.PackFormat.INTERLEAVED)
                    wb16[buf, r, pl.ds(off // 2, _LANES)] = plsc.bitcast(
                        packed, jnp.int32)
                return carry
            lax.fori_loop(0, _INDIM // (8 * _LANES), _pk, 0)

        pending[buf] = pltpu.async_copy(
            wb16.at[buf], wout_hbm.at[pl.ds(row_local, _CHUNK_ROWS)],
            sems.at[buf])

    for p in pending:
        if p is not None:
            p.wait()


def _build_w_half(ei_flat, weight, half):
    mesh = plsc.VectorSubcoreMesh(core_axis_name="c", subcore_axis_name="s")

    def body(ei_hbm, w_hbm, wout_hbm, wbufs, srcbuf, wvbuf, wb16, sems):
        _scatter_body(ei_hbm, w_hbm, wout_hbm, wbufs, srcbuf, wvbuf, wb16,
                      sems, row0=half * _HALF_ROWS)

    k = pl.kernel(
        body,
        mesh=mesh,
        out_type=jax.ShapeDtypeStruct((_HALF_ROWS, _INDIM // 2), jnp.int32),
        scratch_types=[
            pltpu.VMEM((_NBUF, _CHUNK_ROWS, _INDIM), jnp.float32),
            pltpu.VMEM((_NBUF * _CHUNK_EDGES,), jnp.int32),
            pltpu.VMEM((_NBUF * _CHUNK_EDGES,), jnp.float32),
            pltpu.VMEM((_NBUF, _CHUNK_ROWS, _INDIM // 2), jnp.int32),
            pltpu.SemaphoreType.DMA((_NBUF,)),
        ],
        compiler_params=pltpu.CompilerParams(needs_layout_passes=False),
    )
    return k(ei_flat, weight)


_BN = 256


def _as_bf16(w_i32):
    return lax.bitcast_convert_type(w_i32, jnp.bfloat16).reshape(
        _HALF_ROWS, _INDIM)


def _mm_body(x_ref, w_ref, b_ref, o_ref):
    acc = lax.dot_general(
        x_ref[...], w_ref[...],
        (((1,), (1,)), ((), ())),
        preferred_element_type=jnp.float32,
    )
    o_ref[...] = acc + b_ref[...]


def _mm_body_acc(prev_ref, x_ref, w_ref, b_ref, o_ref):
    del prev_ref
    _mm_body(x_ref, w_ref, b_ref, o_ref)


def _matmul_half(prev, xb, w_half, bias2d, half):
    off = half * (_HALF_ROWS // _BN)
    grid = (_HALF_ROWS // _BN,)
    common = dict(
        grid=grid,
        out_specs=pl.BlockSpec((_NTOK, _BN), lambda j, off=off: (0, j + off)),
        out_shape=jax.ShapeDtypeStruct((_NTOK, _OUTDIM), jnp.float32),
    )
    in_specs = [
        pl.BlockSpec((_NTOK, _INDIM), lambda j: (0, 0)),
        pl.BlockSpec((_BN, _INDIM), lambda j: (j, 0)),
        pl.BlockSpec((1, _BN), lambda j, off=off: (0, j + off)),
    ]
    if prev is None:
        return pl.pallas_call(
            _mm_body, in_specs=in_specs, **common,
        )(xb, w_half, bias2d)
    return pl.pallas_call(
        _mm_body_acc,
        in_specs=[pl.BlockSpec(memory_space=pl.ANY)] + in_specs,
        input_output_aliases={0: 0},
        **common,
    )(prev, xb, w_half, bias2d)


@jax.jit
def kernel(x, weight, bias, edge_index):
    xb = x.astype(jnp.bfloat16)
    xb = xb.reshape(_NTOK, _INDIM // 32, 2, 16).transpose(0, 1, 3, 2)
    xb = xb.reshape(_NTOK, _INDIM)
    ei_flat = edge_index.reshape(2 * _E)
    bias2d = bias.reshape(1, _OUTDIM)
    w0 = _as_bf16(_build_w_half(ei_flat, weight, 0))
    w1 = _as_bf16(_build_w_half(ei_flat, weight, 1))
    out = _matmul_half(None, xb, w0, bias2d, 0)
    out = _matmul_half(out, xb, w1, bias2d, 1)
    return out


# speedup vs baseline: 2.7445x; 2.7445x over previous
import jax
import jax.numpy as jnp
from jax import lax
from jax.experimental import pallas as pl
from jax.experimental.pallas import tpu as pltpu
from jax.experimental.pallas import tpu_sc as plsc

_INDIM = 2048
_OUTDIM = 2048
_FANIN = 32
_NTOK = 2048

_E = _OUTDIM * _FANIN
_NUM_WORKERS = 32
_NHALF = 2
_HALF_ROWS = _OUTDIM // _NHALF
_ROWS_PER_WORKER = _HALF_ROWS // _NUM_WORKERS
_CHUNK_ROWS = 16
_CHUNK_EDGES = _CHUNK_ROWS * _FANIN
_LANES = 16
_NBUF = 2


def _scatter_body(ei_hbm, w_hbm, wout_hbm, wbufs, srcbuf, wvbuf, sems, *,
                  row0):
    wid = lax.axis_index("s") * 2 + lax.axis_index("c")
    iota = lax.iota(jnp.int32, _LANES)
    nchunks = _ROWS_PER_WORKER // _CHUNK_ROWS
    pending = [None] * _NBUF

    zeros16 = jnp.zeros((_LANES,), jnp.float32)
    for buf in range(_NBUF):
        for r in range(_CHUNK_ROWS):
            def _zcol(j, carry, buf=buf, r=r):
                base = j * (_LANES * 8)
                for u in range(8):
                    wbufs[buf, r, pl.ds(base + u * _LANES, _LANES)] = zeros16
                return carry
            lax.fori_loop(0, _INDIM // (_LANES * 8), _zcol, 0)

    for chunk in range(nchunks):
        buf = chunk % _NBUF
        row_local = wid * _ROWS_PER_WORKER + chunk * _CHUNK_ROWS
        edge_base = (row0 + row_local) * _FANIN
        wbuf = wbufs.at[buf]

        if pending[buf] is not None:
            pending[buf].wait()
            pending[buf] = None
            for k in range(_FANIN):
                le = iota * _FANIN + (buf * _CHUNK_EDGES + k)
                src_vec = plsc.load_gather(srcbuf, [le])
                w_vec = plsc.load_gather(wvbuf, [le])
                plsc.addupdate_scatter(wbuf, [iota, src_vec], -w_vec)

        pltpu.sync_copy(ei_hbm.at[pl.ds(_E + edge_base, _CHUNK_EDGES)],
                        srcbuf.at[pl.ds(buf * _CHUNK_EDGES, _CHUNK_EDGES)])
        pltpu.sync_copy(w_hbm.at[pl.ds(edge_base, _CHUNK_EDGES)],
                        wvbuf.at[pl.ds(buf * _CHUNK_EDGES, _CHUNK_EDGES)])

        for k in range(_FANIN):
            le = iota * _FANIN + (buf * _CHUNK_EDGES + k)
            src_vec = plsc.load_gather(srcbuf, [le])
            w_vec = plsc.load_gather(wvbuf, [le])
            plsc.addupdate_scatter(wbuf, [iota, src_vec], w_vec)

        pending[buf] = pltpu.async_copy(
            wbuf, wout_hbm.at[pl.ds(row_local, _CHUNK_ROWS)], sems.at[buf])

    for p in pending:
        if p is not None:
            p.wait()


def _build_w_half(ei_flat, weight, half):
    mesh = plsc.VectorSubcoreMesh(core_axis_name="c", subcore_axis_name="s")

    def body(ei_hbm, w_hbm, wout_hbm, wbufs, srcbuf, wvbuf, sems):
        _scatter_body(ei_hbm, w_hbm, wout_hbm, wbufs, srcbuf, wvbuf, sems,
                      row0=half * _HALF_ROWS)

    k = pl.kernel(
        body,
        mesh=mesh,
        out_type=jax.ShapeDtypeStruct((_HALF_ROWS, _INDIM), jnp.float32),
        scratch_types=[
            pltpu.VMEM((_NBUF, _CHUNK_ROWS, _INDIM), jnp.float32),
            pltpu.VMEM((_NBUF * _CHUNK_EDGES,), jnp.int32),
            pltpu.VMEM((_NBUF * _CHUNK_EDGES,), jnp.float32),
            pltpu.SemaphoreType.DMA((_NBUF,)),
        ],
        compiler_params=pltpu.CompilerParams(needs_layout_passes=False),
    )
    return k(ei_flat, weight)


_BN = 256


def _mm_body(x_ref, w_ref, b_ref, o_ref):
    acc = lax.dot_general(
        x_ref[...], w_ref[...].astype(jnp.bfloat16),
        (((1,), (1,)), ((), ())),
        preferred_element_type=jnp.float32,
    )
    o_ref[...] = acc + b_ref[...]


def _mm_body_acc(prev_ref, x_ref, w_ref, b_ref, o_ref):
    del prev_ref
    _mm_body(x_ref, w_ref, b_ref, o_ref)


def _matmul_half(prev, xb, w_half, bias2d, half):
    off = half * (_HALF_ROWS // _BN)
    grid = (_HALF_ROWS // _BN,)
    common = dict(
        grid=grid,
        out_specs=pl.BlockSpec((_NTOK, _BN), lambda j, off=off: (0, j + off)),
        out_shape=jax.ShapeDtypeStruct((_NTOK, _OUTDIM), jnp.float32),
    )
    in_specs = [
        pl.BlockSpec((_NTOK, _INDIM), lambda j: (0, 0)),
        pl.BlockSpec((_BN, _INDIM), lambda j: (j, 0)),
        pl.BlockSpec((1, _BN), lambda j, off=off: (0, j + off)),
    ]
    if prev is None:
        return pl.pallas_call(
            _mm_body, in_specs=in_specs, **common,
        )(xb, w_half, bias2d)
    return pl.pallas_call(
        _mm_body_acc,
        in_specs=[pl.BlockSpec(memory_space=pl.ANY)] + in_specs,
        input_output_aliases={0: 0},
        **common,
    )(prev, xb, w_half, bias2d)


@jax.jit
def kernel(x, weight, bias, edge_index):
    xb = x.astype(jnp.bfloat16)
    ei_flat = edge_index.reshape(2 * _E)
    bias2d = bias.reshape(1, _OUTDIM)
    w0 = _build_w_half(ei_flat, weight, 0)
    w1 = _build_w_half(ei_flat, weight, 1)
    out = _matmul_half(None, xb, w0, bias2d, 0)
    out = _matmul_half(out, xb, w1, bias2d, 1)
    return out
